# Initial kernel scaffold; baseline (speedup 1.0000x reference)
import jax
import jax.numpy as jnp
from jax.experimental import pallas as pl


def kernel(x, edge_index, W, b):
    raise NotImplementedError("write your pallas kernel here")


# speedup vs baseline: 14.9664x; 14.9664x over previous
import dataclasses
import functools

import jax
import jax.numpy as jnp
from jax import lax
from jax.experimental import pallas as pl
from jax.experimental.pallas import tpu as pltpu
from jax.experimental.pallas import tpu_sc as plsc

NC = 2
NS = 16
NW = NC * NS


def _mesh():
    return plsc.VectorSubcoreMesh(core_axis_name="c", subcore_axis_name="s")


def _sc_params():
    cp = pltpu.CompilerParams()
    if "needs_layout_passes" in pltpu.CompilerParams.__dataclass_fields__:
        cp = dataclasses.replace(cp, needs_layout_passes=False)
    return cp


def _degrees(src, dst, n):
    e = src.shape[0]
    assert e % NW == 0
    ew = e // NW
    assert ew % 16 == 0 and n % 16 == 0

    @functools.partial(
        pl.kernel,
        out_type=(jax.ShapeDtypeStruct((NW, n), jnp.float32),
                  jax.ShapeDtypeStruct((NW, n), jnp.float32)),
        mesh=_mesh(),
        compiler_params=_sc_params(),
        scratch_types=[
            pltpu.VMEM((ew,), jnp.int32),
            pltpu.VMEM((ew,), jnp.int32),
            pltpu.VMEM((n,), jnp.float32),
            pltpu.VMEM((n,), jnp.float32),
            pltpu.SemaphoreType.DMA,
        ],
    )
    def deg_kernel(src_hbm, dst_hbm, od_hbm, id_hbm, src_v, dst_v, od_v, id_v,
                   sem):
        cid = lax.axis_index("c")
        sid = lax.axis_index("s")
        wid = cid * NS + sid
        base = wid * ew
        pltpu.async_copy(src_hbm.at[pl.ds(base, ew)], src_v, sem).wait()
        pltpu.async_copy(dst_hbm.at[pl.ds(base, ew)], dst_v, sem).wait()

        zero16 = jnp.zeros((16,), jnp.float32)

        @pl.loop(0, n, step=16)
        def _(i):
            od_v[pl.ds(i, 16)] = zero16
            id_v[pl.ds(i, 16)] = zero16

        ones16 = jnp.ones((16,), jnp.float32)

        @pl.loop(0, ew, step=16)
        def _(i):
            s = src_v[pl.ds(i, 16)]
            plsc.addupdate_scatter(od_v, [s], ones16)
            d = dst_v[pl.ds(i, 16)]
            plsc.addupdate_scatter(id_v, [d], ones16)

        pltpu.async_copy(od_v, od_hbm.at[wid], sem).wait()
        pltpu.async_copy(id_v, id_hbm.at[wid], sem).wait()

    return deg_kernel(src, dst)


def _matmul(x, w):
    n, d_in = x.shape
    d_out = w.shape[0]
    bn = 2000
    assert n % bn == 0

    def body(x_ref, w_ref, o_ref):
        o_ref[...] = lax.dot_general(
            x_ref[...], w_ref[...],
            dimension_numbers=(((1,), (1,)), ((), ())),
            preferred_element_type=jnp.float32)

    return pl.pallas_call(
        body,
        grid=(n // bn,),
        in_specs=[pl.BlockSpec((bn, d_in), lambda i: (i, 0)),
                  pl.BlockSpec((d_out, d_in), lambda i: (0, 0))],
        out_specs=pl.BlockSpec((bn, d_out), lambda i: (i, 0)),
        out_shape=jax.ShapeDtypeStruct((n, d_out), jnp.float32),
    )(x, w)


def _scale(xw, odp_t):
    n, d = xw.shape
    bn = 2000
    assert n % bn == 0

    def body(xw_ref, od_ref, o_ref):
        od = jnp.sum(od_ref[...], axis=1, keepdims=True)
        ci = lax.rsqrt(jnp.maximum(od, 1.0))
        o_ref[...] = xw_ref[...] * ci

    return pl.pallas_call(
        body,
        grid=(n // bn,),
        in_specs=[pl.BlockSpec((bn, d), lambda i: (i, 0)),
                  pl.BlockSpec((bn, NW), lambda i: (i, 0))],
        out_specs=pl.BlockSpec((bn, d), lambda i: (i, 0)),
        out_shape=jax.ShapeDtypeStruct((n, d), jnp.float32),
    )(xw, odp_t)


def _aggregate(xs, src, dst, z):
    n, d = xs.shape
    e = src.shape[0]
    ew = e // NW
    ch = 128
    nch = ew // ch
    tail = ew - nch * ch
    assert tail % 8 == 0
    rps = (n // NS) // 8 * 8
    rtail = n - rps * NS
    assert rtail % 8 == 0

    scratch = [
        pltpu.VMEM_SHARED((n, d), jnp.float32),
        pltpu.VMEM((ch,), jnp.int32),
        pltpu.VMEM((ch,), jnp.int32),
        pltpu.VMEM((ch, d), jnp.float32),
        pltpu.SemaphoreType.DMA,
    ]
    if tail:
        scratch += [
            pltpu.VMEM((tail,), jnp.int32),
            pltpu.VMEM((tail,), jnp.int32),
            pltpu.VMEM((tail, d), jnp.float32),
        ]

    @functools.partial(
        pl.kernel,
        out_type=jax.ShapeDtypeStruct((NC, n, d), jnp.float32),
        mesh=_mesh(),
        compiler_params=_sc_params(),
        scratch_types=scratch,
    )
    def agg_kernel(xs_hbm, src_hbm, dst_hbm, z_hbm, out_hbm, agg_sh, src_v,
                   dst_v, rows_v, sem, *tl):
        cid = lax.axis_index("c")
        sid = lax.axis_index("s")
        wid = cid * NS + sid
        base = wid * ew

        pltpu.sync_copy(z_hbm.at[pl.ds(0, rps)], agg_sh.at[pl.ds(sid * rps, rps)])
        if rtail:
            @pl.when(sid == 0)
            def _():
                pltpu.sync_copy(z_hbm.at[pl.ds(0, rtail)],
                                agg_sh.at[pl.ds(rps * NS, rtail)])
        plsc.subcore_barrier()

        @pl.loop(0, nch)
        def _(i):
            off = base + i * ch
            pltpu.sync_copy(src_hbm.at[pl.ds(off, ch)], src_v)
            pltpu.sync_copy(dst_hbm.at[pl.ds(off, ch)], dst_v)
            pltpu.async_copy(xs_hbm.at[src_v], rows_v, sem).wait()
            pltpu.sync_copy(rows_v, agg_sh.at[dst_v], add=True)

        if tail:
            src_t, dst_t, rows_t = tl
            off = base + nch * ch
            pltpu.sync_copy(src_hbm.at[pl.ds(off, tail)], src_t)
            pltpu.sync_copy(dst_hbm.at[pl.ds(off, tail)], dst_t)
            pltpu.async_copy(xs_hbm.at[src_t], rows_t, sem).wait()
            pltpu.sync_copy(rows_t, agg_sh.at[dst_t], add=True)

        plsc.subcore_barrier()
        pltpu.sync_copy(agg_sh.at[pl.ds(sid * rps, rps)],
                        out_hbm.at[cid].at[pl.ds(sid * rps, rps)])
        if rtail:
            @pl.when(sid == 0)
            def _():
                pltpu.sync_copy(agg_sh.at[pl.ds(rps * NS, rtail)],
                                out_hbm.at[cid].at[pl.ds(rps * NS, rtail)])

    return agg_kernel(xs, src, dst, z)


def _combine(aggp, xw, idp_t, b2):
    n, d = xw.shape
    bn = 2000
    assert n % bn == 0

    def body(agg_ref, xw_ref, id_ref, b_ref, o_ref):
        s = agg_ref[0] + agg_ref[1]
        ind = jnp.sum(id_ref[...], axis=1, keepdims=True)
        cj = lax.rsqrt(jnp.maximum(ind, 1.0))
        keep = jnp.where(ind > 0.0, 0.0, xw_ref[...])
        o_ref[...] = jnp.maximum(s * cj + keep + b_ref[...], 0.0)

    return pl.pallas_call(
        body,
        grid=(n // bn,),
        in_specs=[pl.BlockSpec((NC, bn, d), lambda i: (0, i, 0)),
                  pl.BlockSpec((bn, d), lambda i: (i, 0)),
                  pl.BlockSpec((bn, NW), lambda i: (i, 0)),
                  pl.BlockSpec((1, d), lambda i: (0, 0))],
        out_specs=pl.BlockSpec((bn, d), lambda i: (i, 0)),
        out_shape=jax.ShapeDtypeStruct((n, d), jnp.float32),
    )(aggp, xw, idp_t, b2)


def kernel(x, edge_index, W, b):
    n, d_in = x.shape
    d_out = W.shape[0]
    ei = edge_index.astype(jnp.int32)
    src = ei[0]
    dst = ei[1]

    odp, idp = _degrees(src, dst, n)
    xw = _matmul(x, W)
    xs = _scale(xw, odp.T)
    z = jnp.zeros((n // NS, d_out), jnp.float32)
    aggp = _aggregate(xs, src, dst, z)
    return _combine(aggp, xw, idp.T, b.reshape(1, d_out))


# speedup vs baseline: 23.5275x; 1.5720x over previous
import dataclasses
import functools

import jax
import jax.numpy as jnp
from jax import lax
from jax.experimental import pallas as pl
from jax.experimental.pallas import tpu as pltpu
from jax.experimental.pallas import tpu_sc as plsc

NC = 2
NS = 16
NW = NC * NS


def _mesh():
    return plsc.VectorSubcoreMesh(core_axis_name="c", subcore_axis_name="s")


def _sc_params():
    cp = pltpu.CompilerParams()
    if "needs_layout_passes" in pltpu.CompilerParams.__dataclass_fields__:
        cp = dataclasses.replace(cp, needs_layout_passes=False)
    return cp


def _degrees(src, dst, n):
    e = src.shape[0]
    assert e % NW == 0
    ew = e // NW
    assert ew % 16 == 0 and n % 16 == 0

    @functools.partial(
        pl.kernel,
        out_type=(jax.ShapeDtypeStruct((NW, n), jnp.float32),
                  jax.ShapeDtypeStruct((NW, n), jnp.float32)),
        mesh=_mesh(),
        compiler_params=_sc_params(),
        scratch_types=[
            pltpu.VMEM((ew,), jnp.int32),
            pltpu.VMEM((ew,), jnp.int32),
            pltpu.VMEM((n,), jnp.float32),
            pltpu.VMEM((n,), jnp.float32),
            pltpu.SemaphoreType.DMA,
        ],
    )
    def deg_kernel(src_hbm, dst_hbm, od_hbm, id_hbm, src_v, dst_v, od_v, id_v,
                   sem):
        cid = lax.axis_index("c")
        sid = lax.axis_index("s")
        wid = cid * NS + sid
        base = wid * ew
        pltpu.async_copy(src_hbm.at[pl.ds(base, ew)], src_v, sem).wait()
        pltpu.async_copy(dst_hbm.at[pl.ds(base, ew)], dst_v, sem).wait()

        zero16 = jnp.zeros((16,), jnp.float32)

        @pl.loop(0, n, step=16)
        def _(i):
            od_v[pl.ds(i, 16)] = zero16
            id_v[pl.ds(i, 16)] = zero16

        ones16 = jnp.ones((16,), jnp.float32)

        @pl.loop(0, ew, step=16)
        def _(i):
            s = src_v[pl.ds(i, 16)]
            plsc.addupdate_scatter(od_v, [s], ones16)
            d = dst_v[pl.ds(i, 16)]
            plsc.addupdate_scatter(id_v, [d], ones16)

        pltpu.async_copy(od_v, od_hbm.at[wid], sem).wait()
        pltpu.async_copy(id_v, id_hbm.at[wid], sem).wait()

    return deg_kernel(src, dst)


def _matmul(x, w):
    n, d_in = x.shape
    d_out = w.shape[0]
    bn = 2000
    assert n % bn == 0

    def body(x_ref, w_ref, o_ref):
        o_ref[...] = lax.dot_general(
            x_ref[...], w_ref[...],
            dimension_numbers=(((1,), (1,)), ((), ())),
            preferred_element_type=jnp.float32)

    return pl.pallas_call(
        body,
        grid=(n // bn,),
        in_specs=[pl.BlockSpec((bn, d_in), lambda i: (i, 0)),
                  pl.BlockSpec((d_out, d_in), lambda i: (0, 0))],
        out_specs=pl.BlockSpec((bn, d_out), lambda i: (i, 0)),
        out_shape=jax.ShapeDtypeStruct((n, d_out), jnp.float32),
    )(x, w)


def _scale(xw, odp_t):
    n, d = xw.shape
    bn = 2000
    assert n % bn == 0

    def body(xw_ref, od_ref, o_ref):
        od = jnp.sum(od_ref[...], axis=1, keepdims=True)
        ci = lax.rsqrt(jnp.maximum(od, 1.0))
        o_ref[...] = xw_ref[...] * ci

    return pl.pallas_call(
        body,
        grid=(n // bn,),
        in_specs=[pl.BlockSpec((bn, d), lambda i: (i, 0)),
                  pl.BlockSpec((bn, NW), lambda i: (i, 0))],
        out_specs=pl.BlockSpec((bn, d), lambda i: (i, 0)),
        out_shape=jax.ShapeDtypeStruct((n, d), jnp.float32),
    )(xw, odp_t)


def _aggregate(xs, src, dst, z):
    n, d = xs.shape
    e = src.shape[0]
    ew = e // NW
    ch = 128
    nch = ew // ch
    tail = ew - nch * ch
    assert tail % 8 == 0
    rps = (n // NS) // 8 * 8
    rtail = n - rps * NS
    assert rtail % 8 == 0

    assert nch >= 2 and nch % 2 == 0

    scratch = [
        pltpu.VMEM_SHARED((n, d), jnp.float32),
        pltpu.VMEM((ch,), jnp.int32), pltpu.VMEM((ch,), jnp.int32),
        pltpu.VMEM((ch, d), jnp.float32),
        pltpu.VMEM((ch,), jnp.int32), pltpu.VMEM((ch,), jnp.int32),
        pltpu.VMEM((ch, d), jnp.float32),
        pltpu.SemaphoreType.DMA, pltpu.SemaphoreType.DMA,
        pltpu.SemaphoreType.DMA, pltpu.SemaphoreType.DMA,
    ]
    if tail:
        scratch += [
            pltpu.VMEM((tail,), jnp.int32),
            pltpu.VMEM((tail,), jnp.int32),
            pltpu.VMEM((tail, d), jnp.float32),
        ]

    @functools.partial(
        pl.kernel,
        out_type=jax.ShapeDtypeStruct((NC, n, d), jnp.float32),
        mesh=_mesh(),
        compiler_params=_sc_params(),
        scratch_types=scratch,
    )
    def agg_kernel(xs_hbm, src_hbm, dst_hbm, z_hbm, out_hbm, agg_sh,
                   src_a, dst_a, rows_a, src_b, dst_b, rows_b,
                   sem_ga, sem_gb, sem_ia, sem_ib, *tl):
        cid = lax.axis_index("c")
        sid = lax.axis_index("s")
        wid = cid * NS + sid
        base = wid * ew

        def idx_start(i, src_v, dst_v, sem):
            off = base + i * ch
            pltpu.make_async_copy(src_hbm.at[pl.ds(off, ch)], src_v, sem).start()
            pltpu.make_async_copy(dst_hbm.at[pl.ds(off, ch)], dst_v, sem).start()

        def idx_wait(src_v, dst_v, sem):
            pltpu.make_async_copy(src_hbm.at[pl.ds(0, ch)], src_v, sem).wait()
            pltpu.make_async_copy(dst_hbm.at[pl.ds(0, ch)], dst_v, sem).wait()

        def gather_start(src_v, rows_v, sem):
            pltpu.make_async_copy(xs_hbm.at[src_v], rows_v, sem).start()

        def gather_wait(src_v, rows_v, sem):
            pltpu.make_async_copy(xs_hbm.at[src_v], rows_v, sem).wait()

        def scatter_add(rows_v, dst_v):
            pltpu.sync_copy(rows_v, agg_sh.at[dst_v], add=True)

        pltpu.sync_copy(z_hbm.at[pl.ds(0, rps)], agg_sh.at[pl.ds(sid * rps, rps)])
        if rtail:
            @pl.when(sid == 0)
            def _():
                pltpu.sync_copy(z_hbm.at[pl.ds(0, rtail)],
                                agg_sh.at[pl.ds(rps * NS, rtail)])
        plsc.subcore_barrier()

        idx_start(0, src_a, dst_a, sem_ia)
        idx_wait(src_a, dst_a, sem_ia)
        gather_start(src_a, rows_a, sem_ga)
        idx_start(1, src_b, dst_b, sem_ib)

        @pl.loop(0, nch, step=2)
        def _(g):
            gather_wait(src_a, rows_a, sem_ga)
            idx_wait(src_b, dst_b, sem_ib)
            gather_start(src_b, rows_b, sem_gb)
            scatter_add(rows_a, dst_a)

            @pl.when(g + 2 < nch)
            def _():
                idx_start(g + 2, src_a, dst_a, sem_ia)

            gather_wait(src_b, rows_b, sem_gb)

            @pl.when(g + 2 < nch)
            def _():
                idx_wait(src_a, dst_a, sem_ia)
                gather_start(src_a, rows_a, sem_ga)

            scatter_add(rows_b, dst_b)

            @pl.when(g + 3 < nch)
            def _():
                idx_start(g + 3, src_b, dst_b, sem_ib)

        if tail:
            src_t, dst_t, rows_t = tl
            off = base + nch * ch
            pltpu.sync_copy(src_hbm.at[pl.ds(off, tail)], src_t)
            pltpu.sync_copy(dst_hbm.at[pl.ds(off, tail)], dst_t)
            pltpu.async_copy(xs_hbm.at[src_t], rows_t, sem_ga).wait()
            pltpu.sync_copy(rows_t, agg_sh.at[dst_t], add=True)

        plsc.subcore_barrier()
        pltpu.sync_copy(agg_sh.at[pl.ds(sid * rps, rps)],
                        out_hbm.at[cid].at[pl.ds(sid * rps, rps)])
        if rtail:
            @pl.when(sid == 0)
            def _():
                pltpu.sync_copy(agg_sh.at[pl.ds(rps * NS, rtail)],
                                out_hbm.at[cid].at[pl.ds(rps * NS, rtail)])

    return agg_kernel(xs, src, dst, z)


def _combine(aggp, xw, idp_t, b2):
    n, d = xw.shape
    bn = 2000
    assert n % bn == 0

    def body(agg_ref, xw_ref, id_ref, b_ref, o_ref):
        s = agg_ref[0] + agg_ref[1]
        ind = jnp.sum(id_ref[...], axis=1, keepdims=True)
        cj = lax.rsqrt(jnp.maximum(ind, 1.0))
        keep = jnp.where(ind > 0.0, 0.0, xw_ref[...])
        o_ref[...] = jnp.maximum(s * cj + keep + b_ref[...], 0.0)

    return pl.pallas_call(
        body,
        grid=(n // bn,),
        in_specs=[pl.BlockSpec((NC, bn, d), lambda i: (0, i, 0)),
                  pl.BlockSpec((bn, d), lambda i: (i, 0)),
                  pl.BlockSpec((bn, NW), lambda i: (i, 0)),
                  pl.BlockSpec((1, d), lambda i: (0, 0))],
        out_specs=pl.BlockSpec((bn, d), lambda i: (i, 0)),
        out_shape=jax.ShapeDtypeStruct((n, d), jnp.float32),
    )(aggp, xw, idp_t, b2)


def kernel(x, edge_index, W, b):
    n, d_in = x.shape
    d_out = W.shape[0]
    ei = edge_index.astype(jnp.int32)
    src = ei[0]
    dst = ei[1]

    odp, idp = _degrees(src, dst, n)
    xw = _matmul(x, W)
    xs = _scale(xw, odp.T)
    z = jnp.zeros((n // NS, d_out), jnp.float32)
    aggp = _aggregate(xs, src, dst, z)
    return _combine(aggp, xw, idp.T, b.reshape(1, d_out))


# speedup vs baseline: 25.9566x; 1.1032x over previous
import dataclasses
import functools

import jax
import jax.numpy as jnp
from jax import lax
from jax.experimental import pallas as pl
from jax.experimental.pallas import tpu as pltpu
from jax.experimental.pallas import tpu_sc as plsc

NC = 2
NS = 16
NW = NC * NS


def _mesh():
    return plsc.VectorSubcoreMesh(core_axis_name="c", subcore_axis_name="s")


def _sc_params():
    cp = pltpu.CompilerParams()
    if "needs_layout_passes" in pltpu.CompilerParams.__dataclass_fields__:
        cp = dataclasses.replace(cp, needs_layout_passes=False)
    return cp


def _degrees(src, dst, n):
    e = src.shape[0]
    assert e % NW == 0
    ew = e // NW
    assert ew % 16 == 0 and n % 16 == 0

    @functools.partial(
        pl.kernel,
        out_type=(jax.ShapeDtypeStruct((NW, n), jnp.float32),
                  jax.ShapeDtypeStruct((NW, n), jnp.float32)),
        mesh=_mesh(),
        compiler_params=_sc_params(),
        scratch_types=[
            pltpu.VMEM((ew,), jnp.int32),
            pltpu.VMEM((ew,), jnp.int32),
            pltpu.VMEM((n,), jnp.float32),
            pltpu.VMEM((n,), jnp.float32),
            pltpu.SemaphoreType.DMA,
        ],
    )
    def deg_kernel(src_hbm, dst_hbm, od_hbm, id_hbm, src_v, dst_v, od_v, id_v,
                   sem):
        cid = lax.axis_index("c")
        sid = lax.axis_index("s")
        wid = cid * NS + sid
        base = wid * ew
        pltpu.async_copy(src_hbm.at[pl.ds(base, ew)], src_v, sem).wait()
        pltpu.async_copy(dst_hbm.at[pl.ds(base, ew)], dst_v, sem).wait()

        zero16 = jnp.zeros((16,), jnp.float32)

        @pl.loop(0, n, step=16)
        def _(i):
            od_v[pl.ds(i, 16)] = zero16
            id_v[pl.ds(i, 16)] = zero16

        ones16 = jnp.ones((16,), jnp.float32)

        @pl.loop(0, ew, step=16)
        def _(i):
            s = src_v[pl.ds(i, 16)]
            plsc.addupdate_scatter(od_v, [s], ones16)
            d = dst_v[pl.ds(i, 16)]
            plsc.addupdate_scatter(id_v, [d], ones16)

        pltpu.async_copy(od_v, od_hbm.at[wid], sem).wait()
        pltpu.async_copy(id_v, id_hbm.at[wid], sem).wait()

    return deg_kernel(src, dst)


def _matmul(x, w):
    n, d_in = x.shape
    d_out = w.shape[0]
    bn = 2000
    assert n % bn == 0

    def body(x_ref, w_ref, o_ref):
        o_ref[...] = lax.dot_general(
            x_ref[...], w_ref[...],
            dimension_numbers=(((1,), (1,)), ((), ())),
            preferred_element_type=jnp.float32)

    return pl.pallas_call(
        body,
        grid=(n // bn,),
        in_specs=[pl.BlockSpec((bn, d_in), lambda i: (i, 0)),
                  pl.BlockSpec((d_out, d_in), lambda i: (0, 0))],
        out_specs=pl.BlockSpec((bn, d_out), lambda i: (i, 0)),
        out_shape=jax.ShapeDtypeStruct((n, d_out), jnp.float32),
    )(x, w)


def _scale(xw, odp_t):
    n, d = xw.shape
    bn = 2000
    assert n % bn == 0

    def body(xw_ref, od_ref, o_ref):
        od = jnp.sum(od_ref[...], axis=1, keepdims=True)
        ci = lax.rsqrt(jnp.maximum(od, 1.0))
        o_ref[...] = xw_ref[...] * ci

    return pl.pallas_call(
        body,
        grid=(n // bn,),
        in_specs=[pl.BlockSpec((bn, d), lambda i: (i, 0)),
                  pl.BlockSpec((bn, NW), lambda i: (i, 0))],
        out_specs=pl.BlockSpec((bn, d), lambda i: (i, 0)),
        out_shape=jax.ShapeDtypeStruct((n, d), jnp.float32),
    )(xw, odp_t)


def _aggregate(xs, src, dst, z):
    n, d = xs.shape
    e = src.shape[0]
    ew = e // NW
    budget_words = 2_000_000 - n * d
    ch = None
    for cand in range(128, 0, -8):
        ncand = ew // cand
        t = ew - ncand * cand
        scratch_words = NS * 3 * (cand * d + 2 * cand + 64)
        if (ncand % 3 == 0 and t <= 128 and t % 8 == 0
                and scratch_words <= budget_words):
            ch = cand
            break
    assert ch is not None
    nch = ew // ch
    tail = ew - nch * ch
    rps = (n // NS) // 8 * 8
    rtail = n - rps * NS
    assert rtail % 8 == 0

    assert nch >= 3 and nch % 3 == 0

    scratch = [
        pltpu.VMEM_SHARED((n, d), jnp.float32),
        pltpu.VMEM((ch,), jnp.int32), pltpu.VMEM((ch,), jnp.int32),
        pltpu.VMEM((ch, d), jnp.float32),
        pltpu.VMEM((ch,), jnp.int32), pltpu.VMEM((ch,), jnp.int32),
        pltpu.VMEM((ch, d), jnp.float32),
        pltpu.VMEM((ch,), jnp.int32), pltpu.VMEM((ch,), jnp.int32),
        pltpu.VMEM((ch, d), jnp.float32),
        pltpu.SemaphoreType.DMA, pltpu.SemaphoreType.DMA,
        pltpu.SemaphoreType.DMA, pltpu.SemaphoreType.DMA,
        pltpu.SemaphoreType.DMA, pltpu.SemaphoreType.DMA,
    ]
    if tail:
        scratch += [
            pltpu.VMEM((tail,), jnp.int32),
            pltpu.VMEM((tail,), jnp.int32),
            pltpu.VMEM((tail, d), jnp.float32),
        ]

    @functools.partial(
        pl.kernel,
        out_type=jax.ShapeDtypeStruct((NC, n, d), jnp.float32),
        mesh=_mesh(),
        compiler_params=_sc_params(),
        scratch_types=scratch,
    )
    def agg_kernel(xs_hbm, src_hbm, dst_hbm, z_hbm, out_hbm, agg_sh,
                   src_a, dst_a, rows_a, src_b, dst_b, rows_b,
                   src_c, dst_c, rows_c,
                   sem_ga, sem_gb, sem_gc, sem_ia, sem_ib, sem_ic, *tl):
        cid = lax.axis_index("c")
        sid = lax.axis_index("s")
        wid = cid * NS + sid
        base = wid * ew

        def idx_start(i, src_v, dst_v, sem):
            off = base + i * ch
            pltpu.make_async_copy(src_hbm.at[pl.ds(off, ch)], src_v, sem).start()
            pltpu.make_async_copy(dst_hbm.at[pl.ds(off, ch)], dst_v, sem).start()

        def idx_wait(src_v, dst_v, sem):
            pltpu.make_async_copy(src_hbm.at[pl.ds(0, ch)], src_v, sem).wait()
            pltpu.make_async_copy(dst_hbm.at[pl.ds(0, ch)], dst_v, sem).wait()

        def gather_start(src_v, rows_v, sem):
            pltpu.make_async_copy(xs_hbm.at[src_v], rows_v, sem).start()

        def gather_wait(src_v, rows_v, sem):
            pltpu.make_async_copy(xs_hbm.at[src_v], rows_v, sem).wait()

        def scatter_add(rows_v, dst_v):
            pltpu.sync_copy(rows_v, agg_sh.at[dst_v], add=True)

        pltpu.sync_copy(z_hbm.at[pl.ds(0, rps)], agg_sh.at[pl.ds(sid * rps, rps)])
        if rtail:
            @pl.when(sid == 0)
            def _():
                pltpu.sync_copy(z_hbm.at[pl.ds(0, rtail)],
                                agg_sh.at[pl.ds(rps * NS, rtail)])
        plsc.subcore_barrier()

        idx_start(0, src_a, dst_a, sem_ia)
        idx_start(1, src_b, dst_b, sem_ib)
        idx_wait(src_a, dst_a, sem_ia)
        gather_start(src_a, rows_a, sem_ga)
        idx_wait(src_b, dst_b, sem_ib)
        gather_start(src_b, rows_b, sem_gb)
        idx_start(2, src_c, dst_c, sem_ic)

        @pl.loop(0, nch, step=3)
        def _(g):
            gather_wait(src_a, rows_a, sem_ga)
            scatter_add(rows_a, dst_a)
            idx_wait(src_c, dst_c, sem_ic)
            gather_start(src_c, rows_c, sem_gc)

            @pl.when(g + 3 < nch)
            def _():
                idx_start(g + 3, src_a, dst_a, sem_ia)

            gather_wait(src_b, rows_b, sem_gb)
            scatter_add(rows_b, dst_b)

            @pl.when(g + 3 < nch)
            def _():
                idx_wait(src_a, dst_a, sem_ia)
                gather_start(src_a, rows_a, sem_ga)

            @pl.when(g + 4 < nch)
            def _():
                idx_start(g + 4, src_b, dst_b, sem_ib)

            gather_wait(src_c, rows_c, sem_gc)
            scatter_add(rows_c, dst_c)

            @pl.when(g + 4 < nch)
            def _():
                idx_wait(src_b, dst_b, sem_ib)
                gather_start(src_b, rows_b, sem_gb)

            @pl.when(g + 5 < nch)
            def _():
                idx_start(g + 5, src_c, dst_c, sem_ic)

        if tail:
            src_t, dst_t, rows_t = tl
            off = base + nch * ch
            pltpu.sync_copy(src_hbm.at[pl.ds(off, tail)], src_t)
            pltpu.sync_copy(dst_hbm.at[pl.ds(off, tail)], dst_t)
            pltpu.async_copy(xs_hbm.at[src_t], rows_t, sem_ga).wait()
            pltpu.sync_copy(rows_t, agg_sh.at[dst_t], add=True)

        plsc.subcore_barrier()
        pltpu.sync_copy(agg_sh.at[pl.ds(sid * rps, rps)],
                        out_hbm.at[cid].at[pl.ds(sid * rps, rps)])
        if rtail:
            @pl.when(sid == 0)
            def _():
                pltpu.sync_copy(agg_sh.at[pl.ds(rps * NS, rtail)],
                                out_hbm.at[cid].at[pl.ds(rps * NS, rtail)])

    return agg_kernel(xs, src, dst, z)


def _combine(aggp, xw, idp_t, b2):
    n, d = xw.shape
    bn = 2000
    assert n % bn == 0

    def body(agg_ref, xw_ref, id_ref, b_ref, o_ref):
        s = agg_ref[0] + agg_ref[1]
        ind = jnp.sum(id_ref[...], axis=1, keepdims=True)
        cj = lax.rsqrt(jnp.maximum(ind, 1.0))
        keep = jnp.where(ind > 0.0, 0.0, xw_ref[...])
        o_ref[...] = jnp.maximum(s * cj + keep + b_ref[...], 0.0)

    return pl.pallas_call(
        body,
        grid=(n // bn,),
        in_specs=[pl.BlockSpec((NC, bn, d), lambda i: (0, i, 0)),
                  pl.BlockSpec((bn, d), lambda i: (i, 0)),
                  pl.BlockSpec((bn, NW), lambda i: (i, 0)),
                  pl.BlockSpec((1, d), lambda i: (0, 0))],
        out_specs=pl.BlockSpec((bn, d), lambda i: (i, 0)),
        out_shape=jax.ShapeDtypeStruct((n, d), jnp.float32),
    )(aggp, xw, idp_t, b2)


def kernel(x, edge_index, W, b):
    n, d_in = x.shape
    d_out = W.shape[0]
    ei = edge_index.astype(jnp.int32)
    src = ei[0]
    dst = ei[1]

    odp, idp = _degrees(src, dst, n)
    xw = _matmul(x, W)
    xs = _scale(xw, odp.T)
    z = jnp.zeros((n // NS, d_out), jnp.float32)
    aggp = _aggregate(xs, src, dst, z)
    return _combine(aggp, xw, idp.T, b.reshape(1, d_out))
